# Initial kernel scaffold; baseline (speedup 1.0000x reference)
import jax
import jax.numpy as jnp
from jax.experimental import pallas as pl


def kernel(x, edge_index, W1, b1, g1, be1, W2, b2, g2, be2, W3, b3, g3, be3, W4, b4, g4, be4, Wc, bc):
    raise NotImplementedError("write your pallas kernel here")


# speedup vs baseline: 3.5030x; 3.5030x over previous
import functools

import jax
import jax.numpy as jnp
from jax import lax
from jax.experimental import pallas as pl
from jax.experimental.pallas import tpu as pltpu
from jax.experimental.pallas import tpu_sc as plsc

N = 10000
E = 160000
NC, NS = 2, 16
EPB = 80
NB = 128
NBH = NB // 2
WB = 16
NW = NB // WB
EPAD = NS * NB * EPB
ACC_ROWS = 10112
RPS = ACC_ROWS // NS
FC = 128
BM = 1000
NMB = N // BM
EPS = 1e-5


def _deg_body(dst_hbm, ones_hbm, zeros_hbm, out_hbm, idx_v, ones_v, acc):
    core = lax.axis_index("c")
    sub = lax.axis_index("s")
    pltpu.sync_copy(dst_hbm.at[sub], idx_v)
    pltpu.sync_copy(ones_hbm, ones_v)
    pltpu.sync_copy(zeros_hbm, acc.at[pl.ds(sub * RPS, RPS)])
    plsc.subcore_barrier()

    def body(b, carry):
        pltpu.sync_copy(ones_v, acc.at[idx_v.at[b]], add=True)
        return carry

    lax.fori_loop(0, NB, body, 0)
    plsc.subcore_barrier()
    pltpu.sync_copy(acc.at[pl.ds(sub * RPS, RPS)],
                    out_hbm.at[core].at[pl.ds(sub * RPS, RPS)])


_deg_call = functools.partial(
    pl.kernel,
    out_type=jax.ShapeDtypeStruct((NC, ACC_ROWS, FC), jnp.float32),
    mesh=plsc.VectorSubcoreMesh(core_axis_name="c", subcore_axis_name="s",
                                num_cores=NC, num_subcores=NS),
    scratch_types=[
        pltpu.VMEM((NB, EPB), jnp.int32),
        pltpu.VMEM((EPB, FC), jnp.float32),
        pltpu.VMEM_SHARED((ACC_ROWS, FC), jnp.float32),
    ],
)(_deg_body)


def _make_agg(n_chunks):
    cpc = n_chunks // NC

    def body(table_hbm, src_hbm, dst_hbm, zeros_hbm, out_hbm,
             src_v, dst_v, rows0, rows1, sem0, sem1, acc):
        core = lax.axis_index("c")
        sub = lax.axis_index("s")
        src_h = src_hbm.at[sub]
        dst_h = dst_hbm.at[sub]
        for cc in range(cpc):
            chunk = cc * NC + core
            tbl = table_hbm.at[chunk]
            pltpu.sync_copy(zeros_hbm, acc.at[pl.ds(sub * RPS, RPS)])
            plsc.subcore_barrier()

            def wbody(w, carry):
                pltpu.sync_copy(src_h.at[pl.ds(w * WB, WB)], src_v)
                pltpu.sync_copy(dst_h.at[pl.ds(w * WB, WB)], dst_v)
                pltpu.async_copy(tbl.at[src_v.at[0]], rows0, sem0)

                def ebody(i, c2):
                    b0 = i * 2
                    b1 = b0 + 1
                    pltpu.make_async_copy(tbl.at[src_v.at[b0]], rows0, sem0).wait()
                    pltpu.async_copy(tbl.at[src_v.at[b1]], rows1, sem1)
                    pltpu.sync_copy(rows0, acc.at[dst_v.at[b0]], add=True)
                    pltpu.make_async_copy(tbl.at[src_v.at[b1]], rows1, sem1).wait()

                    @pl.when(i < WB // 2 - 1)
                    def _():
                        pltpu.async_copy(tbl.at[src_v.at[b0 + 2]], rows0, sem0)

                    pltpu.sync_copy(rows1, acc.at[dst_v.at[b1]], add=True)
                    return c2

                lax.fori_loop(0, WB // 2, ebody, 0)
                return carry

            lax.fori_loop(0, NW, wbody, 0)
            plsc.subcore_barrier()
            pltpu.sync_copy(acc.at[pl.ds(sub * RPS, RPS)],
                            out_hbm.at[chunk].at[pl.ds(sub * RPS, RPS)])
            plsc.subcore_barrier()

    return functools.partial(
        pl.kernel,
        out_type=jax.ShapeDtypeStruct((n_chunks, ACC_ROWS, FC), jnp.float32),
        mesh=plsc.VectorSubcoreMesh(core_axis_name="c", subcore_axis_name="s",
                                    num_cores=NC, num_subcores=NS),
        scratch_types=[
            pltpu.VMEM((WB, EPB), jnp.int32),
            pltpu.VMEM((WB, EPB), jnp.int32),
            pltpu.VMEM((EPB, FC), jnp.float32),
            pltpu.VMEM((EPB, FC), jnp.float32),
            pltpu.SemaphoreType.DMA,
            pltpu.SemaphoreType.DMA,
            pltpu.VMEM_SHARED((ACC_ROWS, FC), jnp.float32),
        ],
    )(body)


_agg_calls = {c: _make_agg(c) for c in (2, 4, 8)}


def _dis(cnt_blk):
    return lax.rsqrt(cnt_blk + 1.0)


def _elu(x):
    return jnp.where(x > 0, x, jnp.exp(x) - 1.0)


def _xs_body(cnt_ref, x_ref, xs_ref):
    d = _dis(cnt_ref[...])
    xs_ref[0] = d * x_ref[...]


def _xs_call(cnt, x, n_chunks):
    return pl.pallas_call(
        _xs_body,
        grid=(NMB, n_chunks),
        in_specs=[
            pl.BlockSpec((BM, 1), lambda i, c: (i, 0)),
            pl.BlockSpec((BM, FC), lambda i, c: (i, c)),
        ],
        out_specs=pl.BlockSpec((1, BM, FC), lambda i, c: (c, i, 0)),
        out_shape=jax.ShapeDtypeStruct((n_chunks, N, FC), jnp.float32),
    )(cnt, x)


def _mm_bn_body(nk, agg_ref, xs_ref, cnt_ref, w_ref, b_ref, h_ref, st_ref):
    k = pl.program_id(2)
    i = pl.program_id(1)
    d = _dis(cnt_ref[...])
    a = d * (agg_ref[0] + xs_ref[0])
    partial = jnp.dot(a, w_ref[...], preferred_element_type=jnp.float32)

    @pl.when(k == 0)
    def _():
        h_ref[...] = partial

    @pl.when(k > 0)
    def _():
        h_ref[...] += partial

    @pl.when(k == nk - 1)
    def _():
        h = h_ref[...] + b_ref[...]
        h_ref[...] = h
        s1 = jnp.sum(h, axis=0, keepdims=True)
        s2 = jnp.sum(h * h, axis=0, keepdims=True)

        @pl.when(i == 0)
        def _():
            st_ref[0:1, :] = s1
            st_ref[1:2, :] = s2

        @pl.when(i > 0)
        def _():
            st_ref[0:1, :] += s1
            st_ref[1:2, :] += s2


def _mm_bn_call(aggc, xsc, cnt, w, b):
    din, dout = w.shape
    nk = din // FC
    bn = 512
    nj = dout // bn
    return pl.pallas_call(
        functools.partial(_mm_bn_body, nk),
        grid=(nj, NMB, nk),
        in_specs=[
            pl.BlockSpec((1, BM, FC), lambda j, i, k: (k, i, 0)),
            pl.BlockSpec((1, BM, FC), lambda j, i, k: (k, i, 0)),
            pl.BlockSpec((BM, 1), lambda j, i, k: (i, 0)),
            pl.BlockSpec((FC, bn), lambda j, i, k: (k, j)),
            pl.BlockSpec((1, bn), lambda j, i, k: (0, j)),
        ],
        out_specs=[
            pl.BlockSpec((BM, bn), lambda j, i, k: (i, j)),
            pl.BlockSpec((8, bn), lambda j, i, k: (0, j)),
        ],
        out_shape=[
            jax.ShapeDtypeStruct((N, dout), jnp.float32),
            jax.ShapeDtypeStruct((8, dout), jnp.float32),
        ],
    )(aggc, xsc, cnt, w, b)


def _bn_finalize(h_ref, st_ref, g_ref, be_ref):
    mu = st_ref[0:1, :] / N
    var = st_ref[1:2, :] / N - mu * mu
    rs = lax.rsqrt(var + EPS)
    return g_ref[...] * (h_ref[...] - mu) * rs + be_ref[...]


def _fin1_body(h_ref, st_ref, g_ref, be_ref, cnt_ref, h1_ref, s2_ref):
    a = _elu(_bn_finalize(h_ref, st_ref, g_ref, be_ref))
    h1_ref[...] = a
    d = _dis(cnt_ref[...])
    for c in range(8):
        s2_ref[c] = d * a[:, c * FC:(c + 1) * FC]


def _fin1_call(h, st, g, be, cnt):
    return pl.pallas_call(
        _fin1_body,
        grid=(NMB,),
        in_specs=[
            pl.BlockSpec((BM, 1024), lambda i: (i, 0)),
            pl.BlockSpec((8, 1024), lambda i: (0, 0)),
            pl.BlockSpec((1, 1024), lambda i: (0, 0)),
            pl.BlockSpec((1, 1024), lambda i: (0, 0)),
            pl.BlockSpec((BM, 1), lambda i: (i, 0)),
        ],
        out_specs=[
            pl.BlockSpec((BM, 1024), lambda i: (i, 0)),
            pl.BlockSpec((8, BM, FC), lambda i: (0, i, 0)),
        ],
        out_shape=[
            jax.ShapeDtypeStruct((N, 1024), jnp.float32),
            jax.ShapeDtypeStruct((8, N, FC), jnp.float32),
        ],
    )(h, st, g, be, cnt)


def _fin2_body(h_ref, st_ref, g_ref, be_ref, out_ref):
    out_ref[...] = _elu(_bn_finalize(h_ref, st_ref, g_ref, be_ref))


def _fin2_call(h, st, g, be):
    dout = h.shape[1]
    return pl.pallas_call(
        _fin2_body,
        grid=(NMB,),
        in_specs=[
            pl.BlockSpec((BM, dout), lambda i: (i, 0)),
            pl.BlockSpec((8, dout), lambda i: (0, 0)),
            pl.BlockSpec((1, dout), lambda i: (0, 0)),
            pl.BlockSpec((1, dout), lambda i: (0, 0)),
        ],
        out_specs=pl.BlockSpec((BM, dout), lambda i: (i, 0)),
        out_shape=jax.ShapeDtypeStruct((N, dout), jnp.float32),
    )(h, st, g, be)


def _mm_sc_body(nk, a_ref, cnt_ref, w_ref, ms_ref, acc_ref):
    k = pl.program_id(2)
    partial = jnp.dot(a_ref[...], w_ref[...], preferred_element_type=jnp.float32)

    @pl.when(k == 0)
    def _():
        acc_ref[...] = partial

    @pl.when(k > 0)
    def _():
        acc_ref[...] += partial

    @pl.when(k == nk - 1)
    def _():
        d = _dis(cnt_ref[...])
        m = d * acc_ref[...]
        for c in range(4):
            ms_ref[c] = m[:, c * FC:(c + 1) * FC]


def _mm_sc_call(a, cnt, w):
    din, dout = w.shape
    nk = din // FC
    bn = 512
    nj = dout // bn
    return pl.pallas_call(
        functools.partial(_mm_sc_body, nk),
        grid=(nj, NMB, nk),
        in_specs=[
            pl.BlockSpec((BM, FC), lambda j, i, k: (i, k)),
            pl.BlockSpec((BM, 1), lambda j, i, k: (i, 0)),
            pl.BlockSpec((FC, bn), lambda j, i, k: (k, j)),
        ],
        out_specs=pl.BlockSpec((4, BM, FC), lambda j, i, k: (j, i, 0)),
        out_shape=jax.ShapeDtypeStruct((dout // FC, N, FC), jnp.float32),
        scratch_shapes=[pltpu.VMEM((BM, bn), jnp.float32)],
    )(a, cnt, w)


def _post_body(nc_, agg_ref, ms_ref, cnt_ref, b_ref, conv_ref, st_ref):
    i = pl.program_id(0)
    d = _dis(cnt_ref[...])
    for c in range(nc_):
        conv_ref[:, c * FC:(c + 1) * FC] = (
            d * (agg_ref[c] + ms_ref[c]) + b_ref[:, c * FC:(c + 1) * FC])
    h = conv_ref[...]
    s1 = jnp.sum(h, axis=0, keepdims=True)
    s2 = jnp.sum(h * h, axis=0, keepdims=True)

    @pl.when(i == 0)
    def _():
        st_ref[0:1, :] = s1
        st_ref[1:2, :] = s2

    @pl.when(i > 0)
    def _():
        st_ref[0:1, :] += s1
        st_ref[1:2, :] += s2


def _post_call(aggc, msc, cnt, b):
    nc_ = aggc.shape[0]
    dout = nc_ * FC
    return pl.pallas_call(
        functools.partial(_post_body, nc_),
        grid=(NMB,),
        in_specs=[
            pl.BlockSpec((nc_, BM, FC), lambda i: (0, i, 0)),
            pl.BlockSpec((nc_, BM, FC), lambda i: (0, i, 0)),
            pl.BlockSpec((BM, 1), lambda i: (i, 0)),
            pl.BlockSpec((1, dout), lambda i: (0, 0)),
        ],
        out_specs=[
            pl.BlockSpec((BM, dout), lambda i: (i, 0)),
            pl.BlockSpec((8, dout), lambda i: (0, 0)),
        ],
        out_shape=[
            jax.ShapeDtypeStruct((N, dout), jnp.float32),
            jax.ShapeDtypeStruct((8, dout), jnp.float32),
        ],
    )(aggc, msc, cnt, b)


def _fin3_body(conv_ref, st_ref, g_ref, be_ref, h1_ref, out_ref):
    out_ref[...] = _elu(_bn_finalize(conv_ref, st_ref, g_ref, be_ref)) + h1_ref[...]


def _fin3_call(conv, st, g, be, h1):
    return pl.pallas_call(
        _fin3_body,
        grid=(NMB,),
        in_specs=[
            pl.BlockSpec((BM, 1024), lambda i: (i, 0)),
            pl.BlockSpec((8, 1024), lambda i: (0, 0)),
            pl.BlockSpec((1, 1024), lambda i: (0, 0)),
            pl.BlockSpec((1, 1024), lambda i: (0, 0)),
            pl.BlockSpec((BM, 1024), lambda i: (i, 0)),
        ],
        out_specs=pl.BlockSpec((BM, 1024), lambda i: (i, 0)),
        out_shape=jax.ShapeDtypeStruct((N, 1024), jnp.float32),
    )(conv, st, g, be, h1)


def _cls_body(conv_ref, st_ref, g_ref, be_ref, wc_ref, bc_ref, bn4_ref, log_ref):
    bn4 = _bn_finalize(conv_ref, st_ref, g_ref, be_ref)
    bn4_ref[...] = bn4
    a = _elu(bn4)
    log_ref[...] = jnp.dot(a, wc_ref[...],
                           preferred_element_type=jnp.float32) + bc_ref[...]


def _cls_call(conv, st, g, be, wc, bc):
    return pl.pallas_call(
        _cls_body,
        grid=(NMB,),
        in_specs=[
            pl.BlockSpec((BM, 512), lambda i: (i, 0)),
            pl.BlockSpec((8, 512), lambda i: (0, 0)),
            pl.BlockSpec((1, 512), lambda i: (0, 0)),
            pl.BlockSpec((1, 512), lambda i: (0, 0)),
            pl.BlockSpec((512, 64), lambda i: (0, 0)),
            pl.BlockSpec((1, 64), lambda i: (0, 0)),
        ],
        out_specs=[
            pl.BlockSpec((BM, 512), lambda i: (i, 0)),
            pl.BlockSpec((BM, 64), lambda i: (i, 0)),
        ],
        out_shape=[
            jax.ShapeDtypeStruct((N, 512), jnp.float32),
            jax.ShapeDtypeStruct((N, 64), jnp.float32),
        ],
    )(conv, st, g, be, wc, bc)


def kernel(x, edge_index, W1, b1, g1, be1, W2, b2, g2, be2,
           W3, b3, g3, be3, W4, b4, g4, be4, Wc, bc):
    src = jnp.concatenate(
        [edge_index[0], jnp.zeros((EPAD - E,), jnp.int32)]).reshape(NS, NB, EPB)
    dst = jnp.concatenate(
        [edge_index[1], jnp.full((EPAD - E,), N, jnp.int32)]).reshape(NS, NB, EPB)

    onesfc = jnp.ones((EPB, FC), jnp.float32)
    zerosfc = jnp.zeros((RPS, FC), jnp.float32)

    deg_out = _deg_call(dst, onesfc, zerosfc)
    cnt = lax.slice(deg_out, (0, 0, 0), (1, N, 1)).reshape(N, 1)
    del deg_out

    b1r, b2r, b3r, b4r, bcr = (v.reshape(1, -1) for v in (b1, b2, b3, b4, bc))
    g1r, g2r, g3r, g4r = (v.reshape(1, -1) for v in (g1, g2, g3, g4))
    be1r, be2r, be3r, be4r = (v.reshape(1, -1) for v in (be1, be2, be3, be4))

    xs1 = _xs_call(cnt, x, 2)
    agg1 = _agg_calls[2](xs1, src, dst, zerosfc)
    h1_pre, st1 = _mm_bn_call(agg1, xs1, cnt, W1, b1r)
    h1, s2c = _fin1_call(h1_pre, st1, g1r, be1r, cnt)

    agg2 = _agg_calls[8](s2c, src, dst, zerosfc)
    h2_pre, st2 = _mm_bn_call(agg2, s2c, cnt, W2, b2r)
    h2 = _fin2_call(h2_pre, st2, g2r, be2r)

    ms3 = _mm_sc_call(h2, cnt, W3)
    agg3 = _agg_calls[8](ms3, src, dst, zerosfc)
    conv3, st3 = _post_call(agg3, ms3, cnt, b3r)
    h4_in = _fin3_call(conv3, st3, g3r, be3r, h1)

    ms4 = _mm_sc_call(h4_in, cnt, W4)
    agg4 = _agg_calls[4](ms4, src, dst, zerosfc)
    out_conv4, st4 = _post_call(agg4, ms4, cnt, b4r)

    out_bn4, logits = _cls_call(out_conv4, st4, g4r, be4r, Wc, bcr)
    return (logits, out_conv4, out_bn4)


# speedup vs baseline: 3.9781x; 1.1356x over previous
import functools

import jax
import jax.numpy as jnp
from jax import lax
from jax.experimental import pallas as pl
from jax.experimental.pallas import tpu as pltpu
from jax.experimental.pallas import tpu_sc as plsc

N = 10000
E = 160000
NC, NS = 2, 16
EPB = 80
NB = 128
NBH = NB // 2
WB = 32
NW = NB // WB
EPAD = NS * NB * EPB
ACC_ROWS = 10112
RPS = ACC_ROWS // NS
FC = 128
BM = 1000
NMB = N // BM
EPS = 1e-5


def _deg_body(dst_hbm, ones_hbm, zeros_hbm, out_hbm, idx_v, ones_v, acc):
    core = lax.axis_index("c")
    sub = lax.axis_index("s")
    pltpu.sync_copy(dst_hbm.at[sub], idx_v)
    pltpu.sync_copy(ones_hbm, ones_v)
    pltpu.sync_copy(zeros_hbm, acc.at[pl.ds(sub * RPS, RPS)])
    plsc.subcore_barrier()

    def body(b, carry):
        pltpu.sync_copy(ones_v, acc.at[idx_v.at[b]], add=True)
        return carry

    lax.fori_loop(0, NB, body, 0)
    plsc.subcore_barrier()
    pltpu.sync_copy(acc.at[pl.ds(sub * RPS, RPS)],
                    out_hbm.at[core].at[pl.ds(sub * RPS, RPS)])


_deg_call = functools.partial(
    pl.kernel,
    out_type=jax.ShapeDtypeStruct((NC, ACC_ROWS, FC), jnp.float32),
    mesh=plsc.VectorSubcoreMesh(core_axis_name="c", subcore_axis_name="s",
                                num_cores=NC, num_subcores=NS),
    scratch_types=[
        pltpu.VMEM((NB, EPB), jnp.int32),
        pltpu.VMEM((EPB, FC), jnp.float32),
        pltpu.VMEM_SHARED((ACC_ROWS, FC), jnp.float32),
    ],
)(_deg_body)


def _make_agg(n_chunks):
    cpc = n_chunks // NC

    def body(table_hbm, src_hbm, dst_hbm, zeros_hbm, out_hbm,
             src_v, dst_v, rows0, rows1, rows2, rows3,
             sem0, sem1, sem2, sem3, acc):
        core = lax.axis_index("c")
        sub = lax.axis_index("s")
        rows = (rows0, rows1, rows2, rows3)
        sems = (sem0, sem1, sem2, sem3)
        src_h = src_hbm.at[sub]
        dst_h = dst_hbm.at[sub]
        for cc in range(cpc):
            chunk = cc * NC + core
            tbl = table_hbm.at[chunk]
            pltpu.sync_copy(zeros_hbm, acc.at[pl.ds(sub * RPS, RPS)])
            plsc.subcore_barrier()

            def wbody(w, carry):
                pltpu.sync_copy(src_h.at[pl.ds(w * WB, WB)], src_v)
                pltpu.sync_copy(dst_h.at[pl.ds(w * WB, WB)], dst_v)
                for p in range(4):
                    pltpu.async_copy(tbl.at[src_v.at[p]], rows[p], sems[p])

                def ebody(i, c2):
                    for p in range(4):
                        b = i * 4 + p
                        pltpu.make_async_copy(tbl.at[src_v.at[b]],
                                              rows[p], sems[p]).wait()
                        pltpu.sync_copy(rows[p], acc.at[dst_v.at[b]], add=True)

                        @pl.when(b + 4 < WB)
                        def _():
                            pltpu.async_copy(tbl.at[src_v.at[b + 4]],
                                             rows[p], sems[p])

                    return c2

                lax.fori_loop(0, WB // 4, ebody, 0)
                return carry

            lax.fori_loop(0, NW, wbody, 0)
            plsc.subcore_barrier()
            pltpu.sync_copy(acc.at[pl.ds(sub * RPS, RPS)],
                            out_hbm.at[chunk].at[pl.ds(sub * RPS, RPS)])
            plsc.subcore_barrier()

    return functools.partial(
        pl.kernel,
        out_type=jax.ShapeDtypeStruct((n_chunks, ACC_ROWS, FC), jnp.float32),
        mesh=plsc.VectorSubcoreMesh(core_axis_name="c", subcore_axis_name="s",
                                    num_cores=NC, num_subcores=NS),
        scratch_types=[
            pltpu.VMEM((WB, EPB), jnp.int32),
            pltpu.VMEM((WB, EPB), jnp.int32),
            pltpu.VMEM((EPB, FC), jnp.float32),
            pltpu.VMEM((EPB, FC), jnp.float32),
            pltpu.VMEM((EPB, FC), jnp.float32),
            pltpu.VMEM((EPB, FC), jnp.float32),
            pltpu.SemaphoreType.DMA,
            pltpu.SemaphoreType.DMA,
            pltpu.SemaphoreType.DMA,
            pltpu.SemaphoreType.DMA,
            pltpu.VMEM_SHARED((ACC_ROWS, FC), jnp.float32),
        ],
    )(body)


_agg_calls = {c: _make_agg(c) for c in (2, 4, 8)}


def _dis(cnt_blk):
    return lax.rsqrt(cnt_blk + 1.0)


def _elu(x):
    return jnp.where(x > 0, x, jnp.exp(x) - 1.0)


def _xs_body(cnt_ref, x_ref, xs_ref):
    d = _dis(cnt_ref[...])
    xs_ref[0] = d * x_ref[...]


def _xs_call(cnt, x, n_chunks):
    return pl.pallas_call(
        _xs_body,
        grid=(NMB, n_chunks),
        in_specs=[
            pl.BlockSpec((BM, 1), lambda i, c: (i, 0)),
            pl.BlockSpec((BM, FC), lambda i, c: (i, c)),
        ],
        out_specs=pl.BlockSpec((1, BM, FC), lambda i, c: (c, i, 0)),
        out_shape=jax.ShapeDtypeStruct((n_chunks, N, FC), jnp.float32),
    )(cnt, x)


def _mm_bn_body(nk, agg_ref, xs_ref, cnt_ref, w_ref, b_ref, h_ref, st_ref):
    k = pl.program_id(2)
    i = pl.program_id(1)
    d = _dis(cnt_ref[...])
    a = d * (agg_ref[0] + xs_ref[0])
    partial = jnp.dot(a, w_ref[...], preferred_element_type=jnp.float32)

    @pl.when(k == 0)
    def _():
        h_ref[...] = partial

    @pl.when(k > 0)
    def _():
        h_ref[...] += partial

    @pl.when(k == nk - 1)
    def _():
        h = h_ref[...] + b_ref[...]
        h_ref[...] = h
        s1 = jnp.sum(h, axis=0, keepdims=True)
        s2 = jnp.sum(h * h, axis=0, keepdims=True)

        @pl.when(i == 0)
        def _():
            st_ref[0:1, :] = s1
            st_ref[1:2, :] = s2

        @pl.when(i > 0)
        def _():
            st_ref[0:1, :] += s1
            st_ref[1:2, :] += s2


def _mm_bn_call(aggc, xsc, cnt, w, b):
    din, dout = w.shape
    nk = din // FC
    bn = 512
    nj = dout // bn
    return pl.pallas_call(
        functools.partial(_mm_bn_body, nk),
        grid=(nj, NMB, nk),
        in_specs=[
            pl.BlockSpec((1, BM, FC), lambda j, i, k: (k, i, 0)),
            pl.BlockSpec((1, BM, FC), lambda j, i, k: (k, i, 0)),
            pl.BlockSpec((BM, 1), lambda j, i, k: (i, 0)),
            pl.BlockSpec((FC, bn), lambda j, i, k: (k, j)),
            pl.BlockSpec((1, bn), lambda j, i, k: (0, j)),
        ],
        out_specs=[
            pl.BlockSpec((BM, bn), lambda j, i, k: (i, j)),
            pl.BlockSpec((8, bn), lambda j, i, k: (0, j)),
        ],
        out_shape=[
            jax.ShapeDtypeStruct((N, dout), jnp.float32),
            jax.ShapeDtypeStruct((8, dout), jnp.float32),
        ],
    )(aggc, xsc, cnt, w, b)


def _bn_finalize(h_ref, st_ref, g_ref, be_ref):
    mu = st_ref[0:1, :] / N
    var = st_ref[1:2, :] / N - mu * mu
    rs = lax.rsqrt(var + EPS)
    return g_ref[...] * (h_ref[...] - mu) * rs + be_ref[...]


def _fin1_body(h_ref, st_ref, g_ref, be_ref, cnt_ref, h1_ref, s2_ref):
    a = _elu(_bn_finalize(h_ref, st_ref, g_ref, be_ref))
    h1_ref[...] = a
    d = _dis(cnt_ref[...])
    for c in range(8):
        s2_ref[c] = d * a[:, c * FC:(c + 1) * FC]


def _fin1_call(h, st, g, be, cnt):
    return pl.pallas_call(
        _fin1_body,
        grid=(NMB,),
        in_specs=[
            pl.BlockSpec((BM, 1024), lambda i: (i, 0)),
            pl.BlockSpec((8, 1024), lambda i: (0, 0)),
            pl.BlockSpec((1, 1024), lambda i: (0, 0)),
            pl.BlockSpec((1, 1024), lambda i: (0, 0)),
            pl.BlockSpec((BM, 1), lambda i: (i, 0)),
        ],
        out_specs=[
            pl.BlockSpec((BM, 1024), lambda i: (i, 0)),
            pl.BlockSpec((8, BM, FC), lambda i: (0, i, 0)),
        ],
        out_shape=[
            jax.ShapeDtypeStruct((N, 1024), jnp.float32),
            jax.ShapeDtypeStruct((8, N, FC), jnp.float32),
        ],
    )(h, st, g, be, cnt)


def _fin2_body(h_ref, st_ref, g_ref, be_ref, out_ref):
    out_ref[...] = _elu(_bn_finalize(h_ref, st_ref, g_ref, be_ref))


def _fin2_call(h, st, g, be):
    dout = h.shape[1]
    return pl.pallas_call(
        _fin2_body,
        grid=(NMB,),
        in_specs=[
            pl.BlockSpec((BM, dout), lambda i: (i, 0)),
            pl.BlockSpec((8, dout), lambda i: (0, 0)),
            pl.BlockSpec((1, dout), lambda i: (0, 0)),
            pl.BlockSpec((1, dout), lambda i: (0, 0)),
        ],
        out_specs=pl.BlockSpec((BM, dout), lambda i: (i, 0)),
        out_shape=jax.ShapeDtypeStruct((N, dout), jnp.float32),
    )(h, st, g, be)


def _mm_sc_body(nk, a_ref, cnt_ref, w_ref, ms_ref, acc_ref):
    k = pl.program_id(2)
    partial = jnp.dot(a_ref[...], w_ref[...], preferred_element_type=jnp.float32)

    @pl.when(k == 0)
    def _():
        acc_ref[...] = partial

    @pl.when(k > 0)
    def _():
        acc_ref[...] += partial

    @pl.when(k == nk - 1)
    def _():
        d = _dis(cnt_ref[...])
        m = d * acc_ref[...]
        for c in range(4):
            ms_ref[c] = m[:, c * FC:(c + 1) * FC]


def _mm_sc_call(a, cnt, w):
    din, dout = w.shape
    nk = din // FC
    bn = 512
    nj = dout // bn
    return pl.pallas_call(
        functools.partial(_mm_sc_body, nk),
        grid=(nj, NMB, nk),
        in_specs=[
            pl.BlockSpec((BM, FC), lambda j, i, k: (i, k)),
            pl.BlockSpec((BM, 1), lambda j, i, k: (i, 0)),
            pl.BlockSpec((FC, bn), lambda j, i, k: (k, j)),
        ],
        out_specs=pl.BlockSpec((4, BM, FC), lambda j, i, k: (j, i, 0)),
        out_shape=jax.ShapeDtypeStruct((dout // FC, N, FC), jnp.float32),
        scratch_shapes=[pltpu.VMEM((BM, bn), jnp.float32)],
    )(a, cnt, w)


def _post_body(nc_, agg_ref, ms_ref, cnt_ref, b_ref, conv_ref, st_ref):
    i = pl.program_id(0)
    d = _dis(cnt_ref[...])
    for c in range(nc_):
        conv_ref[:, c * FC:(c + 1) * FC] = (
            d * (agg_ref[c] + ms_ref[c]) + b_ref[:, c * FC:(c + 1) * FC])
    h = conv_ref[...]
    s1 = jnp.sum(h, axis=0, keepdims=True)
    s2 = jnp.sum(h * h, axis=0, keepdims=True)

    @pl.when(i == 0)
    def _():
        st_ref[0:1, :] = s1
        st_ref[1:2, :] = s2

    @pl.when(i > 0)
    def _():
        st_ref[0:1, :] += s1
        st_ref[1:2, :] += s2


def _post_call(aggc, msc, cnt, b):
    nc_ = aggc.shape[0]
    dout = nc_ * FC
    return pl.pallas_call(
        functools.partial(_post_body, nc_),
        grid=(NMB,),
        in_specs=[
            pl.BlockSpec((nc_, BM, FC), lambda i: (0, i, 0)),
            pl.BlockSpec((nc_, BM, FC), lambda i: (0, i, 0)),
            pl.BlockSpec((BM, 1), lambda i: (i, 0)),
            pl.BlockSpec((1, dout), lambda i: (0, 0)),
        ],
        out_specs=[
            pl.BlockSpec((BM, dout), lambda i: (i, 0)),
            pl.BlockSpec((8, dout), lambda i: (0, 0)),
        ],
        out_shape=[
            jax.ShapeDtypeStruct((N, dout), jnp.float32),
            jax.ShapeDtypeStruct((8, dout), jnp.float32),
        ],
    )(aggc, msc, cnt, b)


def _fin3_body(conv_ref, st_ref, g_ref, be_ref, h1_ref, out_ref):
    out_ref[...] = _elu(_bn_finalize(conv_ref, st_ref, g_ref, be_ref)) + h1_ref[...]


def _fin3_call(conv, st, g, be, h1):
    return pl.pallas_call(
        _fin3_body,
        grid=(NMB,),
        in_specs=[
            pl.BlockSpec((BM, 1024), lambda i: (i, 0)),
            pl.BlockSpec((8, 1024), lambda i: (0, 0)),
            pl.BlockSpec((1, 1024), lambda i: (0, 0)),
            pl.BlockSpec((1, 1024), lambda i: (0, 0)),
            pl.BlockSpec((BM, 1024), lambda i: (i, 0)),
        ],
        out_specs=pl.BlockSpec((BM, 1024), lambda i: (i, 0)),
        out_shape=jax.ShapeDtypeStruct((N, 1024), jnp.float32),
    )(conv, st, g, be, h1)


def _cls_body(conv_ref, st_ref, g_ref, be_ref, wc_ref, bc_ref, bn4_ref, log_ref):
    bn4 = _bn_finalize(conv_ref, st_ref, g_ref, be_ref)
    bn4_ref[...] = bn4
    a = _elu(bn4)
    log_ref[...] = jnp.dot(a, wc_ref[...],
                           preferred_element_type=jnp.float32) + bc_ref[...]


def _cls_call(conv, st, g, be, wc, bc):
    return pl.pallas_call(
        _cls_body,
        grid=(NMB,),
        in_specs=[
            pl.BlockSpec((BM, 512), lambda i: (i, 0)),
            pl.BlockSpec((8, 512), lambda i: (0, 0)),
            pl.BlockSpec((1, 512), lambda i: (0, 0)),
            pl.BlockSpec((1, 512), lambda i: (0, 0)),
            pl.BlockSpec((512, 64), lambda i: (0, 0)),
            pl.BlockSpec((1, 64), lambda i: (0, 0)),
        ],
        out_specs=[
            pl.BlockSpec((BM, 512), lambda i: (i, 0)),
            pl.BlockSpec((BM, 64), lambda i: (i, 0)),
        ],
        out_shape=[
            jax.ShapeDtypeStruct((N, 512), jnp.float32),
            jax.ShapeDtypeStruct((N, 64), jnp.float32),
        ],
    )(conv, st, g, be, wc, bc)


def kernel(x, edge_index, W1, b1, g1, be1, W2, b2, g2, be2,
           W3, b3, g3, be3, W4, b4, g4, be4, Wc, bc):
    src = jnp.concatenate(
        [edge_index[0], jnp.zeros((EPAD - E,), jnp.int32)]).reshape(NS, NB, EPB)
    dst = jnp.concatenate(
        [edge_index[1], jnp.full((EPAD - E,), N, jnp.int32)]).reshape(NS, NB, EPB)

    onesfc = jnp.ones((EPB, FC), jnp.float32)
    zerosfc = jnp.zeros((RPS, FC), jnp.float32)

    deg_out = _deg_call(dst, onesfc, zerosfc)
    cnt = lax.slice(deg_out, (0, 0, 0), (1, N, 1)).reshape(N, 1)
    del deg_out

    b1r, b2r, b3r, b4r, bcr = (v.reshape(1, -1) for v in (b1, b2, b3, b4, bc))
    g1r, g2r, g3r, g4r = (v.reshape(1, -1) for v in (g1, g2, g3, g4))
    be1r, be2r, be3r, be4r = (v.reshape(1, -1) for v in (be1, be2, be3, be4))

    xs1 = _xs_call(cnt, x, 2)
    agg1 = _agg_calls[2](xs1, src, dst, zerosfc)
    h1_pre, st1 = _mm_bn_call(agg1, xs1, cnt, W1, b1r)
    h1, s2c = _fin1_call(h1_pre, st1, g1r, be1r, cnt)

    agg2 = _agg_calls[8](s2c, src, dst, zerosfc)
    h2_pre, st2 = _mm_bn_call(agg2, s2c, cnt, W2, b2r)
    h2 = _fin2_call(h2_pre, st2, g2r, be2r)

    ms3 = _mm_sc_call(h2, cnt, W3)
    agg3 = _agg_calls[8](ms3, src, dst, zerosfc)
    conv3, st3 = _post_call(agg3, ms3, cnt, b3r)
    h4_in = _fin3_call(conv3, st3, g3r, be3r, h1)

    ms4 = _mm_sc_call(h4_in, cnt, W4)
    agg4 = _agg_calls[4](ms4, src, dst, zerosfc)
    out_conv4, st4 = _post_call(agg4, ms4, cnt, b4r)

    out_bn4, logits = _cls_call(out_conv4, st4, g4r, be4r, Wc, bcr)
    return (logits, out_conv4, out_bn4)
